# baseline (speedup 1.0000x reference)
import jax
import jax.numpy as jnp
from jax.experimental import pallas as pl
from jax.experimental.pallas import tpu as pltpu

_N = 10000
_F = 128
_BM = 200
_NBUF = 3
_STEPS = _N // _BM
_FULL_ROUNDS = _STEPS // _NBUF
_REM = _STEPS - _FULL_ROUNDS * _NBUF
_KC = 2000


def _body(adj_hbm, x_hbm, w_ref, out_ref, buf_ref, sem, sup_ref, x_sem):
    def _start(step, slot):
        pltpu.make_async_copy(
            adj_hbm.at[pl.ds(step * _BM, _BM), :],
            buf_ref.at[slot],
            sem.at[slot],
        ).start()

    x_copy = pltpu.make_async_copy(x_hbm, sup_ref, x_sem)
    x_copy.start()

    for slot in range(_NBUF):
        _start(slot, slot)

    x_copy.wait()
    for m in range(0, _N, _KC):
        sup_ref[m:m + _KC, :] = jax.lax.dot_general(
            sup_ref[m:m + _KC, :], w_ref[...], (((1,), (0,)), ((), ())),
            preferred_element_type=jnp.float32,
            precision=jax.lax.Precision.HIGHEST)

    def _step(i, slot):
        pltpu.make_async_copy(
            adj_hbm.at[pl.ds(i * _BM, _BM), :],
            buf_ref.at[slot],
            sem.at[slot],
        ).wait()
        acc = jax.lax.dot_general(
            buf_ref[slot], sup_ref[...], (((1,), (0,)), ((), ())),
            preferred_element_type=jnp.float32)
        out_ref[pl.ds(i * _BM, _BM), :] = jnp.maximum(acc, 0.0)

        @pl.when(i + _NBUF < _STEPS)
        def _():
            _start(i + _NBUF, slot)

    def _round(b, carry):
        for slot in range(_NBUF):
            _step(b * _NBUF + slot, slot)
        return carry

    jax.lax.fori_loop(0, _FULL_ROUNDS, _round, 0)

    for slot in range(_REM):
        _step(_FULL_ROUNDS * _NBUF + slot, slot)


def kernel(adj, x_input, weight):
    return pl.pallas_call(
        _body,
        in_specs=[pl.BlockSpec(memory_space=pl.ANY),
                  pl.BlockSpec(memory_space=pl.ANY),
                  pl.BlockSpec((_F, _F), lambda: (0, 0))],
        out_specs=pl.BlockSpec((_N, _F), lambda: (0, 0)),
        out_shape=jax.ShapeDtypeStruct((_N, _F), jnp.float32),
        scratch_shapes=[
            pltpu.VMEM((_NBUF, _BM, _N), jnp.float32),
            pltpu.SemaphoreType.DMA((_NBUF,)),
            pltpu.VMEM((_N, _F), jnp.float32),
            pltpu.SemaphoreType.DMA,
        ],
        compiler_params=pltpu.CompilerParams(
            dimension_semantics=()),
    )(adj, x_input, weight)


# speedup vs baseline: 1.0338x; 1.0338x over previous
import jax
import jax.numpy as jnp
from jax.experimental import pallas as pl
from jax.experimental.pallas import tpu as pltpu

_N = 10000
_F = 128
_BM = 80
_NBUF = 5
_STEPS = _N // _BM


def _body(adj_hbm, x_ref, w_ref, out_ref, buf_ref, sem, sup_ref):
    def _start(step, slot):
        pltpu.make_async_copy(
            adj_hbm.at[pl.ds(step * _BM, _BM), :],
            buf_ref.at[slot],
            sem.at[slot],
        ).start()

    for slot in range(_NBUF):
        _start(slot, slot)

    sup_ref[...] = jax.lax.dot_general(
        x_ref[...], w_ref[...], (((1,), (0,)), ((), ())),
        preferred_element_type=jnp.float32,
        precision=jax.lax.Precision.HIGHEST).astype(jnp.bfloat16)

    def _round(b, carry):
        for slot in range(_NBUF):
            i = b * _NBUF + slot
            pltpu.make_async_copy(
                adj_hbm.at[pl.ds(i * _BM, _BM), :],
                buf_ref.at[slot],
                sem.at[slot],
            ).wait()
            acc = jax.lax.dot_general(
                buf_ref[slot], sup_ref[...], (((1,), (0,)), ((), ())),
                preferred_element_type=jnp.float32)
            out_ref[pl.ds(i * _BM, _BM), :] = jnp.maximum(acc, 0.0)

            @pl.when(i + _NBUF < _STEPS)
            def _():
                _start(i + _NBUF, slot)

        return carry

    jax.lax.fori_loop(0, _STEPS // _NBUF, _round, 0)


def kernel(adj, x_input, weight):
    return pl.pallas_call(
        _body,
        in_specs=[pl.BlockSpec(memory_space=pl.ANY),
                  pl.BlockSpec((_N, _F), lambda: (0, 0)),
                  pl.BlockSpec((_F, _F), lambda: (0, 0))],
        out_specs=pl.BlockSpec((_N, _F), lambda: (0, 0)),
        out_shape=jax.ShapeDtypeStruct((_N, _F), jnp.float32),
        scratch_shapes=[
            pltpu.VMEM((_NBUF, _BM, _N), jnp.float32),
            pltpu.SemaphoreType.DMA((_NBUF,)),
            pltpu.VMEM((_N, _F), jnp.bfloat16),
        ],
        compiler_params=pltpu.CompilerParams(
            dimension_semantics=()),
    )(adj, x_input, weight)
